# baseline (speedup 1.0000x reference)
import jax
import jax.numpy as jnp
from jax import lax
from jax.experimental import pallas as pl
from jax.experimental.pallas import tpu as pltpu
from jax.experimental.pallas import tpu_sc as plsc

BATCH = 4096
MAXLEN = 200
EMBED = 64

_NW = 32
_K = 256
_JPT = BATCH // _K
_NU = MAXLEN * _JPT // _NW
_GK = _K // 128


def _unit_coords(u):
    return u // _JPT, lax.rem(u, _JPT)


def _body(x_hbm, tok_hbm, pos_hbm, out_hbm,
          idx_v, rows_v, tile_v, pos_v, sem_i, sem_g, sem_o):
    wid = lax.axis_index("s") * 2 + lax.axis_index("c")
    u0 = wid * _NU
    iota16 = lax.iota(jnp.int32, 16)

    pltpu.sync_copy(pos_hbm, pos_v)

    def idx_copy(u, b):
        t, j2 = _unit_coords(u)
        return pltpu.make_async_copy(
            x_hbm.at[t, pl.ds(_GK * j2, _GK)], idx_v.at[b], sem_i.at[b])

    def gather_copy(u, b, h):
        return pltpu.make_async_copy(
            tok_hbm.at[idx_v.at[b, h]],
            rows_v.at[b, pl.ds(128 * h, 128)], sem_g.at[b])

    def out_copy(u, b):
        t, j2 = _unit_coords(u)
        return pltpu.make_async_copy(
            tile_v.at[b], out_hbm.at[t, :, pl.ds(_GK * j2, _GK)], sem_o.at[b])

    def compute(u, b):
        t, _ = _unit_coords(u)
        rows2 = rows_v.at[b]
        t16 = jnp.full((16,), t, jnp.int32)

        def per_e(e, carry):
            cols = jnp.full((16,), e, jnp.int32)
            pv = plsc.load_gather(pos_v, [t16, cols])
            i = e // 8
            r = lax.rem(e, 8)

            def per_group(g, c2):
                for gl in range(4):
                    gg = 4 * g + gl
                    rows = iota16 + 16 * gg
                    jl = gg // 8
                    c0 = lax.rem(gg, 8) * 16
                    val = plsc.load_gather(rows2, [rows, cols])
                    tile_v[b, i, jl, r, pl.ds(c0, 16)] = val + pv
                return c2

            lax.fori_loop(0, 4, per_group, 0)
            return carry

        lax.fori_loop(0, EMBED, per_e, 0)

    idx_copy(u0, 0).start()
    idx_copy(u0 + 1, 1).start()
    idx_copy(u0, 0).wait()
    for h in range(_GK):
        gather_copy(u0, 0, h).start()

    def step(p, carry):
        for b in (0, 1):
            u = u0 + 2 * p + b
            un = 2 * p + b
            for h in range(_GK):
                gather_copy(u, b, h).wait()

            @pl.when(un + 1 < _NU)
            def _():
                idx_copy(u + 1, 1 - b).wait()
                for h in range(_GK):
                    gather_copy(u + 1, 1 - b, h).start()

            @pl.when(un + 2 < _NU)
            def _():
                idx_copy(u + 2, b).start()

            @pl.when(un >= 2)
            def _():
                out_copy(u - 2, b).wait()

            compute(u, b)
            out_copy(u, b).start()
        return carry

    lax.fori_loop(0, _NU // 2, step, 0)
    out_copy(u0 + _NU - 2, 0).wait()
    out_copy(u0 + _NU - 1, 1).wait()


@jax.jit
def _tpe(xt, token_table, pos_table):
    mesh = plsc.VectorSubcoreMesh(core_axis_name="c", subcore_axis_name="s")
    return pl.kernel(
        _body,
        out_type=jax.ShapeDtypeStruct(
            (MAXLEN, EMBED // 8, BATCH // 128, 8, 128), jnp.float32),
        mesh=mesh,
        scratch_types=[
            pltpu.VMEM((2, _GK, 128), jnp.int32),
            pltpu.VMEM((2, _K, EMBED), jnp.float32),
            pltpu.VMEM((2, EMBED // 8, _GK, 8, 128), jnp.float32),
            pltpu.VMEM((MAXLEN, EMBED), jnp.float32),
            pltpu.SemaphoreType.DMA((2,)),
            pltpu.SemaphoreType.DMA((2,)),
            pltpu.SemaphoreType.DMA((2,)),
        ],
        compiler_params=pltpu.CompilerParams(
            use_tc_tiling_on_sc=False, needs_layout_passes=False),
    )(xt, token_table, pos_table)


def kernel(x, token_table, pos_table):
    xt = x.astype(jnp.int32).T.reshape(MAXLEN, BATCH // 128, 128)
    out5 = _tpe(xt, token_table, pos_table)
    return out5.transpose(2, 4, 0, 1, 3).reshape(BATCH, MAXLEN, EMBED)


# speedup vs baseline: 1.1559x; 1.1559x over previous
import jax
import jax.numpy as jnp
from jax import lax
from jax.experimental import pallas as pl
from jax.experimental.pallas import tpu as pltpu
from jax.experimental.pallas import tpu_sc as plsc

BATCH = 4096
MAXLEN = 200
EMBED = 64

_NW = 32
_K = 256
_JPT = BATCH // _K
_NU = MAXLEN * _JPT // _NW
_GK = _K // 128


def _unit_coords(u):
    return u // _JPT, lax.rem(u, _JPT)


def _body(x_hbm, tok_hbm, pos_hbm, out_hbm,
          idx_v, rows_v, tile_v, pos_v, sem_i, sem_g, sem_o):
    wid = lax.axis_index("s") * 2 + lax.axis_index("c")
    u0 = wid * _NU
    iota16 = lax.iota(jnp.int32, 16)

    pltpu.sync_copy(pos_hbm, pos_v)

    def idx_copy(u, b):
        t, j2 = _unit_coords(u)
        return pltpu.make_async_copy(
            x_hbm.at[t, pl.ds(_GK * j2, _GK)], idx_v.at[b], sem_i.at[b])

    def gather_copy(u, b, h):
        return pltpu.make_async_copy(
            tok_hbm.at[idx_v.at[b, h]],
            rows_v.at[b, pl.ds(128 * h, 128)], sem_g.at[b])

    def out_copy(u, b):
        t, j2 = _unit_coords(u)
        return pltpu.make_async_copy(
            tile_v.at[b], out_hbm.at[t, :, pl.ds(_GK * j2, _GK)], sem_o.at[b])

    i_idx = [(iota16 + 16 * k) // 8 for k in range(EMBED // 16)]
    r_idx = [lax.rem(iota16 + 16 * k, 8) for k in range(EMBED // 16)]
    jl_idx = [jnp.full((16,), jl, jnp.int32) for jl in range(_GK)]

    def compute(u, b):
        t, _ = _unit_coords(u)
        rows2 = rows_v.at[b]
        pv = [pos_v[t, pl.ds(16 * k, 16)] for k in range(EMBED // 16)]

        dst = tile_v.at[b]

        def per_b(b2, carry):
            c16 = jnp.full((16,), b2, jnp.int32)
            for jl in range(_GK):
                row = 128 * jl + b2
                for k in range(EMBED // 16):
                    val = rows2[row, pl.ds(16 * k, 16)] + pv[k]
                    plsc.store_scatter(
                        dst, [i_idx[k], jl_idx[jl], r_idx[k], c16], val)
            return carry

        lax.fori_loop(0, 128, per_b, 0, unroll=2)

    idx_copy(u0, 0).start()
    idx_copy(u0 + 1, 1).start()
    idx_copy(u0, 0).wait()
    for h in range(_GK):
        gather_copy(u0, 0, h).start()

    def step(p, carry):
        for b in (0, 1):
            u = u0 + 2 * p + b
            un = 2 * p + b
            for h in range(_GK):
                gather_copy(u, b, h).wait()

            @pl.when(un + 1 < _NU)
            def _():
                idx_copy(u + 1, 1 - b).wait()
                for h in range(_GK):
                    gather_copy(u + 1, 1 - b, h).start()

            @pl.when(un + 2 < _NU)
            def _():
                idx_copy(u + 2, b).start()

            @pl.when(un >= 2)
            def _():
                out_copy(u - 2, b).wait()

            compute(u, b)
            out_copy(u, b).start()
        return carry

    lax.fori_loop(0, _NU // 2, step, 0)
    out_copy(u0 + _NU - 2, 0).wait()
    out_copy(u0 + _NU - 1, 1).wait()


@jax.jit
def _tpe(xt, token_table, pos_table):
    mesh = plsc.VectorSubcoreMesh(core_axis_name="c", subcore_axis_name="s")
    return pl.kernel(
        _body,
        out_type=jax.ShapeDtypeStruct(
            (MAXLEN, EMBED // 8, BATCH // 128, 8, 128), jnp.float32),
        mesh=mesh,
        scratch_types=[
            pltpu.VMEM((2, _GK, 128), jnp.int32),
            pltpu.VMEM((2, _K, EMBED), jnp.float32),
            pltpu.VMEM((2, EMBED // 8, _GK, 8, 128), jnp.float32),
            pltpu.VMEM((MAXLEN, EMBED), jnp.float32),
            pltpu.SemaphoreType.DMA((2,)),
            pltpu.SemaphoreType.DMA((2,)),
            pltpu.SemaphoreType.DMA((2,)),
        ],
        compiler_params=pltpu.CompilerParams(
            use_tc_tiling_on_sc=False, needs_layout_passes=False),
    )(xt, token_table, pos_table)


def kernel(x, token_table, pos_table):
    xt = x.astype(jnp.int32).T.reshape(MAXLEN, BATCH // 128, 128)
    out5 = _tpe(xt, token_table, pos_table)
    return out5.transpose(2, 4, 0, 1, 3).reshape(BATCH, MAXLEN, EMBED)


# speedup vs baseline: 1.5221x; 1.3167x over previous
import jax
import jax.numpy as jnp
from jax import lax
from jax.experimental import pallas as pl
from jax.experimental.pallas import tpu as pltpu
from jax.experimental.pallas import tpu_sc as plsc

BATCH = 4096
MAXLEN = 200
EMBED = 64

_NW = 32
_K = 256
_JPT = BATCH // _K
_NU = MAXLEN * _JPT // _NW
_GK = _K // 128


def _unit_coords(u):
    return u // _JPT, lax.rem(u, _JPT)


def _body(x_hbm, tok_hbm, pos_hbm, out_hbm,
          idx_v, rows_v, tile_v, pos_v, sem_i, sem_g, sem_o):
    wid = lax.axis_index("s") * 2 + lax.axis_index("c")
    u0 = wid * _NU
    iota16 = lax.iota(jnp.int32, 16)

    pltpu.sync_copy(pos_hbm, pos_v)

    def idx_copy(u, b):
        t, j2 = _unit_coords(u)
        return pltpu.make_async_copy(
            x_hbm.at[t, pl.ds(_GK * j2, _GK)], idx_v.at[b], sem_i.at[b])

    def gather_copy(u, b, h):
        return pltpu.make_async_copy(
            tok_hbm.at[idx_v.at[b, h]],
            rows_v.at[b, pl.ds(128 * h, 128)], sem_g.at[b])

    def out_copy(u, b):
        t, j2 = _unit_coords(u)
        return pltpu.make_async_copy(
            tile_v.at[b], out_hbm.at[t, :, pl.ds(_GK * j2, _GK)], sem_o.at[b])

    i_idx = [(iota16 + 16 * k) // 8 for k in range(EMBED // 16)]
    r_idx = [lax.rem(iota16 + 16 * k, 8) for k in range(EMBED // 16)]
    jl_idx = [jnp.full((16,), jl, jnp.int32) for jl in range(_GK)]

    def compute(u, b):
        t, _ = _unit_coords(u)
        rows2 = rows_v.at[b]
        pv = [pos_v[t, pl.ds(16 * k, 16)] for k in range(EMBED // 16)]

        dst = tile_v.at[b]

        @plsc.parallel_loop(0, 128, unroll=4)
        def per_b(b2):
            c16 = jnp.full((16,), b2, jnp.int32)
            for jl in range(_GK):
                row = 128 * jl + b2
                for k in range(EMBED // 16):
                    val = rows2[row, pl.ds(16 * k, 16)] + pv[k]
                    plsc.store_scatter(
                        dst, [i_idx[k], jl_idx[jl], r_idx[k], c16], val)

    idx_copy(u0, 0).start()
    idx_copy(u0 + 1, 1).start()
    idx_copy(u0, 0).wait()
    for h in range(_GK):
        gather_copy(u0, 0, h).start()

    def step(p, carry):
        for b in (0, 1):
            u = u0 + 2 * p + b
            un = 2 * p + b
            for h in range(_GK):
                gather_copy(u, b, h).wait()

            @pl.when(un + 1 < _NU)
            def _():
                idx_copy(u + 1, 1 - b).wait()
                for h in range(_GK):
                    gather_copy(u + 1, 1 - b, h).start()

            @pl.when(un + 2 < _NU)
            def _():
                idx_copy(u + 2, b).start()

            @pl.when(un >= 2)
            def _():
                out_copy(u - 2, b).wait()

            compute(u, b)
            out_copy(u, b).start()
        return carry

    lax.fori_loop(0, _NU // 2, step, 0)
    out_copy(u0 + _NU - 2, 0).wait()
    out_copy(u0 + _NU - 1, 1).wait()


@jax.jit
def _tpe(xt, token_table, pos_table):
    mesh = plsc.VectorSubcoreMesh(core_axis_name="c", subcore_axis_name="s")
    return pl.kernel(
        _body,
        out_type=jax.ShapeDtypeStruct(
            (MAXLEN, EMBED // 8, BATCH // 128, 8, 128), jnp.float32),
        mesh=mesh,
        scratch_types=[
            pltpu.VMEM((2, _GK, 128), jnp.int32),
            pltpu.VMEM((2, _K, EMBED), jnp.float32),
            pltpu.VMEM((2, EMBED // 8, _GK, 8, 128), jnp.float32),
            pltpu.VMEM((MAXLEN, EMBED), jnp.float32),
            pltpu.SemaphoreType.DMA((2,)),
            pltpu.SemaphoreType.DMA((2,)),
            pltpu.SemaphoreType.DMA((2,)),
        ],
        compiler_params=pltpu.CompilerParams(
            use_tc_tiling_on_sc=False, needs_layout_passes=False),
    )(xt, token_table, pos_table)


def kernel(x, token_table, pos_table):
    xt = x.astype(jnp.int32).T.reshape(MAXLEN, BATCH // 128, 128)
    out5 = _tpe(xt, token_table, pos_table)
    return out5.transpose(2, 4, 0, 1, 3).reshape(BATCH, MAXLEN, EMBED)
